# baseline (speedup 1.0000x reference)
import dataclasses
import functools

import jax
import jax.numpy as jnp
from jax import lax
from jax.experimental import pallas as pl
from jax.experimental.pallas import tpu as pltpu
from jax.experimental.pallas import tpu_sc as plsc

_WA = 16
_WG = 32
_NWORKERS = 32
_LANES = 16
_HALF = 16
_NIT = 8


def kernel(track_ids, embedding_weight):
    b, s = track_ids.shape
    v, d = embedding_weight.shape
    n = b * s
    per_w = n // _NWORKERS
    ncol = d // _LANES
    a_rows = _NIT * 4 * _WA

    idx = track_ids.reshape(_NWORKERS, per_w).astype(jnp.int32)

    mesh = plsc.VectorSubcoreMesh(
        core_axis_name="core", subcore_axis_name="subcore"
    )
    cp = pltpu.CompilerParams()
    if "needs_layout_passes" in pltpu.CompilerParams.__dataclass_fields__:
        cp = dataclasses.replace(cp, needs_layout_passes=False)

    @functools.partial(
        pl.kernel,
        out_type=jax.ShapeDtypeStruct((n, d), embedding_weight.dtype),
        mesh=mesh,
        compiler_params=cp,
        scratch_types=[
            pltpu.VMEM((v, d), jnp.float32),
            pltpu.VMEM((per_w,), jnp.int32),
            pltpu.VMEM((_WA, d), jnp.float32),
            pltpu.VMEM((_WA, d), jnp.float32),
            pltpu.VMEM((_WG, d), jnp.float32),
            pltpu.VMEM((_WG, d), jnp.float32),
            pltpu.SemaphoreType.DMA,
            pltpu.SemaphoreType.DMA,
            pltpu.SemaphoreType.DMA,
            pltpu.SemaphoreType.DMA,
            pltpu.SemaphoreType.DMA,
            pltpu.SemaphoreType.DMA,
            pltpu.SemaphoreType.DMA,
        ],
    )
    def _expand(
        table_hbm, idx_hbm, out_hbm,
        table_v, idx_v, ob0, ob1, gb0, gb1,
        sem_in, sw0, sw1, sg0, sg1, swg0, swg1,
    ):
        core = lax.axis_index("core")
        sub = lax.axis_index("subcore")
        wid = sub * 2 + core
        pltpu.async_copy(table_hbm, table_v, sem_in).wait()
        pltpu.async_copy(idx_hbm.at[wid], idx_v, sem_in).wait()
        base = wid * per_w
        cols = [lax.iota(jnp.int32, _LANES) + c * _LANES for c in range(ncol)]
        lane = [jnp.full((_LANES,), 0, jnp.int32) + j for j in range(_LANES)]

        def g_issue(gbuf, sgs, gk):
            pltpu.async_copy(
                table_hbm.at[idx_v.at[pl.ds(a_rows + gk * _WG, _WG)]],
                gbuf,
                sgs,
            )

        def g_wait(gbuf, sgs):
            pltpu.make_async_copy(
                table_hbm.at[idx_v.at[pl.ds(0, _WG)]], gbuf, sgs
            ).wait()

        def g_write(gbuf, swgs, gk):
            pltpu.async_copy(
                gbuf,
                out_hbm.at[pl.ds(base + a_rows + gk * _WG, _WG)],
                swgs,
            )

        def gw_wait(gbuf, swgs):
            pltpu.make_async_copy(
                gbuf, out_hbm.at[pl.ds(base, _WG)], swgs
            ).wait()

        def assemble(kk, obuf):
            ids_vec = idx_v[pl.ds(kk * _WA, _LANES)]
            rids = [
                ids_vec.at[lane[j]].get(mode="promise_in_bounds")
                for j in range(_LANES)
            ]
            prev = None
            for j in range(_LANES):
                for half in range(ncol // _HALF):
                    vals = []
                    for u in range(_HALF):
                        c = half * _HALF + u
                        if prev is not None:
                            prow, phalf, pvals = prev
                            pc = phalf * _HALF + u
                            obuf[prow, pl.ds(pc * _LANES, _LANES)] = pvals[u]
                        vals.append(
                            plsc.load_gather(table_v, [rids[j], cols[c]])
                        )
                    prev = (j, half, vals)
            prow, phalf, pvals = prev
            for u in range(_HALF):
                pc = phalf * _HALF + u
                obuf[prow, pl.ds(pc * _LANES, _LANES)] = pvals[u]

        def a_chunk(kk, obuf, sw, first):
            @pl.when(jnp.logical_not(first))
            def _drain():
                pltpu.make_async_copy(
                    obuf, out_hbm.at[pl.ds(base, _WA)], sw
                ).wait()

            assemble(kk, obuf)
            pltpu.async_copy(obuf, out_hbm.at[pl.ds(base + kk * _WA, _WA)], sw)

        g_issue(gb0, sg0, 0)
        g_issue(gb1, sg1, 1)

        @pl.loop(0, _NIT)
        def _round(it):
            first = it == 0
            last = it == _NIT - 1
            g_wait(gb0, sg0)
            g_write(gb0, swg0, 2 * it)
            a_chunk(4 * it + 0, ob0, sw0, first)
            a_chunk(4 * it + 1, ob1, sw1, first)
            g_wait(gb1, sg1)
            g_write(gb1, swg1, 2 * it + 1)

            @pl.when(jnp.logical_not(last))
            def _reissue0():
                gw_wait(gb0, swg0)
                g_issue(gb0, sg0, 2 * it + 2)

            a_chunk(4 * it + 2, ob0, sw0, False)
            a_chunk(4 * it + 3, ob1, sw1, False)

            @pl.when(jnp.logical_not(last))
            def _reissue1():
                gw_wait(gb1, swg1)
                g_issue(gb1, sg1, 2 * it + 3)

        gw_wait(gb0, swg0)
        gw_wait(gb1, swg1)
        for obuf, sw in ((ob0, sw0), (ob1, sw1)):
            pltpu.make_async_copy(obuf, out_hbm.at[pl.ds(base, _WA)], sw).wait()

    return _expand(embedding_weight, idx).reshape(b, s, d)


# speedup vs baseline: 1.2580x; 1.2580x over previous
import dataclasses
import functools

import jax
import jax.numpy as jnp
from jax import lax
from jax.experimental import pallas as pl
from jax.experimental.pallas import tpu as pltpu
from jax.experimental.pallas import tpu_sc as plsc

_WA = 16
_WG = 16
_NWORKERS = 32
_LANES = 16
_HALF = 16
_NIT = 8


def kernel(track_ids, embedding_weight):
    b, s = track_ids.shape
    v, d = embedding_weight.shape
    n = b * s
    per_w = n // _NWORKERS
    ncol = d // _LANES
    a_rows = _NIT * 6 * _WA

    idx = track_ids.reshape(_NWORKERS, per_w).astype(jnp.int32)

    mesh = plsc.VectorSubcoreMesh(
        core_axis_name="core", subcore_axis_name="subcore"
    )
    cp = pltpu.CompilerParams()
    if "needs_layout_passes" in pltpu.CompilerParams.__dataclass_fields__:
        cp = dataclasses.replace(cp, needs_layout_passes=False)

    @functools.partial(
        pl.kernel,
        out_type=jax.ShapeDtypeStruct((n, d), embedding_weight.dtype),
        mesh=mesh,
        compiler_params=cp,
        scratch_types=[
            pltpu.VMEM((v, d), jnp.float32),
            pltpu.VMEM((per_w,), jnp.int32),
            pltpu.VMEM((_WA, d), jnp.float32),
            pltpu.VMEM((_WA, d), jnp.float32),
            pltpu.VMEM((_WG, d), jnp.float32),
            pltpu.VMEM((_WG, d), jnp.float32),
            pltpu.SemaphoreType.DMA,
            pltpu.SemaphoreType.DMA,
            pltpu.SemaphoreType.DMA,
            pltpu.SemaphoreType.DMA,
            pltpu.SemaphoreType.DMA,
            pltpu.SemaphoreType.DMA,
            pltpu.SemaphoreType.DMA,
        ],
    )
    def _expand(
        table_hbm, idx_hbm, out_hbm,
        table_v, idx_v, ob0, ob1, gb0, gb1,
        sem_in, sw0, sw1, sg0, sg1, swg0, swg1,
    ):
        core = lax.axis_index("core")
        sub = lax.axis_index("subcore")
        wid = sub * 2 + core
        pltpu.async_copy(table_hbm, table_v, sem_in).wait()
        pltpu.async_copy(idx_hbm.at[wid], idx_v, sem_in).wait()
        base = wid * per_w
        cols = [lax.iota(jnp.int32, _LANES) + c * _LANES for c in range(ncol)]
        lane = [jnp.full((_LANES,), 0, jnp.int32) + j for j in range(_LANES)]

        def g_issue(gbuf, sgs, gk):
            pltpu.async_copy(
                table_hbm.at[idx_v.at[pl.ds(a_rows + gk * _WG, _WG)]],
                gbuf,
                sgs,
            )

        def g_wait(gbuf, sgs):
            pltpu.make_async_copy(
                table_hbm.at[idx_v.at[pl.ds(0, _WG)]], gbuf, sgs
            ).wait()

        def g_write(gbuf, swgs, gk):
            pltpu.async_copy(
                gbuf,
                out_hbm.at[pl.ds(base + a_rows + gk * _WG, _WG)],
                swgs,
            )

        def gw_wait(gbuf, swgs):
            pltpu.make_async_copy(
                gbuf, out_hbm.at[pl.ds(base, _WG)], swgs
            ).wait()

        def assemble(kk, obuf):
            ids_vec = idx_v[pl.ds(kk * _WA, _LANES)]
            rids = [
                ids_vec.at[lane[j]].get(mode="promise_in_bounds")
                for j in range(_LANES)
            ]
            prev = None
            for j in range(_LANES):
                for half in range(ncol // _HALF):
                    vals = []
                    for u in range(_HALF):
                        c = half * _HALF + u
                        if prev is not None:
                            prow, phalf, pvals = prev
                            pc = phalf * _HALF + u
                            obuf[prow, pl.ds(pc * _LANES, _LANES)] = pvals[u]
                        vals.append(
                            plsc.load_gather(table_v, [rids[j], cols[c]])
                        )
                    prev = (j, half, vals)
            prow, phalf, pvals = prev
            for u in range(_HALF):
                pc = phalf * _HALF + u
                obuf[prow, pl.ds(pc * _LANES, _LANES)] = pvals[u]

        def a_chunk(kk, obuf, sw, first):
            @pl.when(jnp.logical_not(first))
            def _drain():
                pltpu.make_async_copy(
                    obuf, out_hbm.at[pl.ds(base, _WA)], sw
                ).wait()

            assemble(kk, obuf)
            pltpu.async_copy(obuf, out_hbm.at[pl.ds(base + kk * _WA, _WA)], sw)

        g_issue(gb0, sg0, 0)
        g_issue(gb1, sg1, 1)

        @pl.loop(0, _NIT)
        def _round(it):
            first = it == 0
            last = it == _NIT - 1
            g_wait(gb0, sg0)
            g_write(gb0, swg0, 2 * it)
            a_chunk(6 * it + 0, ob0, sw0, first)
            a_chunk(6 * it + 1, ob1, sw1, first)
            a_chunk(6 * it + 2, ob0, sw0, False)
            g_wait(gb1, sg1)
            g_write(gb1, swg1, 2 * it + 1)

            @pl.when(jnp.logical_not(last))
            def _reissue0():
                gw_wait(gb0, swg0)
                g_issue(gb0, sg0, 2 * it + 2)

            a_chunk(6 * it + 3, ob1, sw1, False)
            a_chunk(6 * it + 4, ob0, sw0, False)
            a_chunk(6 * it + 5, ob1, sw1, False)

            @pl.when(jnp.logical_not(last))
            def _reissue1():
                gw_wait(gb1, swg1)
                g_issue(gb1, sg1, 2 * it + 3)

        gw_wait(gb0, swg0)
        gw_wait(gb1, swg1)
        for obuf, sw in ((ob0, sw0), (ob1, sw1)):
            pltpu.make_async_copy(obuf, out_hbm.at[pl.ds(base, _WA)], sw).wait()

    return _expand(embedding_weight, idx).reshape(b, s, d)


# speedup vs baseline: 1.4537x; 1.1555x over previous
import dataclasses
import functools

import jax
import jax.numpy as jnp
from jax import lax
from jax.experimental import pallas as pl
from jax.experimental.pallas import tpu as pltpu
from jax.experimental.pallas import tpu_sc as plsc

_W = 32
_NWORKERS = 32
_LANES = 16
_HALF = 8


def kernel(track_ids, embedding_weight):
    b, s = track_ids.shape
    v, d = embedding_weight.shape
    n = b * s
    per_w = n // _NWORKERS
    nchunk = per_w // _W
    ngroup = _W // _LANES
    ncol = d // _LANES

    idx = track_ids.reshape(_NWORKERS, per_w).astype(jnp.int32)

    mesh = plsc.VectorSubcoreMesh(
        core_axis_name="core", subcore_axis_name="subcore"
    )
    cp = pltpu.CompilerParams()
    if "needs_layout_passes" in pltpu.CompilerParams.__dataclass_fields__:
        cp = dataclasses.replace(cp, needs_layout_passes=False)

    @functools.partial(
        pl.kernel,
        out_type=jax.ShapeDtypeStruct((n, d), embedding_weight.dtype),
        mesh=mesh,
        compiler_params=cp,
        scratch_types=[
            pltpu.VMEM((v, d), jnp.float32),
            pltpu.VMEM((per_w,), jnp.int32),
            pltpu.VMEM((_W, d), jnp.float32),
            pltpu.VMEM((_W, d), jnp.float32),
            pltpu.SemaphoreType.DMA,
            pltpu.SemaphoreType.DMA,
            pltpu.SemaphoreType.DMA,
        ],
    )
    def _expand(
        table_hbm, idx_hbm, out_hbm, table_v, idx_v, ob0, ob1, sem_in, sw0, sw1
    ):
        core = lax.axis_index("core")
        sub = lax.axis_index("subcore")
        wid = sub * 2 + core
        pltpu.async_copy(table_hbm, table_v, sem_in).wait()
        pltpu.async_copy(idx_hbm.at[wid], idx_v, sem_in).wait()
        base = wid * per_w
        cols = [lax.iota(jnp.int32, _LANES) + c * _LANES for c in range(ncol)]
        lane = [jnp.full((_LANES,), 0, jnp.int32) + j for j in range(_LANES)]

        def assemble(kk, obuf):
            rids = []
            prev = None
            for g in range(ngroup):
                ids_vec = idx_v[pl.ds(kk * _W + g * _LANES, _LANES)]
                rids = [
                    ids_vec.at[lane[j]].get(mode="promise_in_bounds")
                    for j in range(_LANES)
                ]
                for j in range(_LANES):
                    row = g * _LANES + j
                    for half in range(ncol // _HALF):
                        vals = []
                        for u in range(_HALF):
                            c = half * _HALF + u
                            if prev is not None:
                                prow, phalf, pvals = prev
                                pc = phalf * _HALF + u
                                obuf[prow, pl.ds(pc * _LANES, _LANES)] = pvals[u]
                            vals.append(
                                plsc.load_gather(table_v, [rids[j], cols[c]])
                            )
                        prev = (row, half, vals)
            prow, phalf, pvals = prev
            for u in range(_HALF):
                pc = phalf * _HALF + u
                obuf[prow, pl.ds(pc * _LANES, _LANES)] = pvals[u]

        @pl.loop(0, nchunk, step=2)
        def _chunks(k0):
            for bslot, (obuf, sw) in enumerate(((ob0, sw0), (ob1, sw1))):
                kk = k0 + bslot

                @pl.when(k0 > 0)
                def _drain():
                    pltpu.make_async_copy(
                        obuf, out_hbm.at[pl.ds(base, _W)], sw
                    ).wait()

                assemble(kk, obuf)
                pltpu.async_copy(
                    obuf, out_hbm.at[pl.ds(base + kk * _W, _W)], sw
                )

        for obuf, sw in ((ob0, sw0), (ob1, sw1)):
            pltpu.make_async_copy(obuf, out_hbm.at[pl.ds(base, _W)], sw).wait()

    return _expand(embedding_weight, idx).reshape(b, s, d)


# speedup vs baseline: 1.5629x; 1.0751x over previous
import dataclasses
import functools

import jax
import jax.numpy as jnp
from jax import lax
from jax.experimental import pallas as pl
from jax.experimental.pallas import tpu as pltpu
from jax.experimental.pallas import tpu_sc as plsc

_W = 32
_NWORKERS = 32
_LANES = 16
_HALF = 16


def kernel(track_ids, embedding_weight):
    b, s = track_ids.shape
    v, d = embedding_weight.shape
    n = b * s
    per_w = n // _NWORKERS
    nchunk = per_w // _W
    ngroup = _W // _LANES
    ncol = d // _LANES

    idx = track_ids.reshape(_NWORKERS, per_w).astype(jnp.int32)

    mesh = plsc.VectorSubcoreMesh(
        core_axis_name="core", subcore_axis_name="subcore"
    )
    cp = pltpu.CompilerParams()
    if "needs_layout_passes" in pltpu.CompilerParams.__dataclass_fields__:
        cp = dataclasses.replace(cp, needs_layout_passes=False)

    @functools.partial(
        pl.kernel,
        out_type=jax.ShapeDtypeStruct((n, d), embedding_weight.dtype),
        mesh=mesh,
        compiler_params=cp,
        scratch_types=[
            pltpu.VMEM((v, d), jnp.float32),
            pltpu.VMEM((per_w,), jnp.int32),
            pltpu.VMEM((_W, d), jnp.float32),
            pltpu.VMEM((_W, d), jnp.float32),
            pltpu.SemaphoreType.DMA,
            pltpu.SemaphoreType.DMA,
            pltpu.SemaphoreType.DMA,
        ],
    )
    def _expand(
        table_hbm, idx_hbm, out_hbm, table_v, idx_v, ob0, ob1, sem_in, sw0, sw1
    ):
        core = lax.axis_index("core")
        sub = lax.axis_index("subcore")
        wid = sub * 2 + core
        pltpu.async_copy(table_hbm, table_v, sem_in).wait()
        pltpu.async_copy(idx_hbm.at[wid], idx_v, sem_in).wait()
        base = wid * per_w
        cols = [lax.iota(jnp.int32, _LANES) + c * _LANES for c in range(ncol)]
        lane = [jnp.full((_LANES,), 0, jnp.int32) + j for j in range(_LANES)]

        def assemble(kk, obuf):
            rids = []
            prev = None
            for g in range(ngroup):
                ids_vec = idx_v[pl.ds(kk * _W + g * _LANES, _LANES)]
                rids = [
                    ids_vec.at[lane[j]].get(mode="promise_in_bounds")
                    for j in range(_LANES)
                ]
                for j in range(_LANES):
                    row = g * _LANES + j
                    for half in range(ncol // _HALF):
                        vals = []
                        for u in range(_HALF):
                            c = half * _HALF + u
                            if prev is not None:
                                prow, phalf, pvals = prev
                                pc = phalf * _HALF + u
                                obuf[prow, pl.ds(pc * _LANES, _LANES)] = pvals[u]
                            vals.append(
                                plsc.load_gather(table_v, [rids[j], cols[c]])
                            )
                        prev = (row, half, vals)
            prow, phalf, pvals = prev
            for u in range(_HALF):
                pc = phalf * _HALF + u
                obuf[prow, pl.ds(pc * _LANES, _LANES)] = pvals[u]

        @pl.loop(0, nchunk, step=2)
        def _chunks(k0):
            for bslot, (obuf, sw) in enumerate(((ob0, sw0), (ob1, sw1))):
                kk = k0 + bslot

                @pl.when(k0 > 0)
                def _drain():
                    pltpu.make_async_copy(
                        obuf, out_hbm.at[pl.ds(base, _W)], sw
                    ).wait()

                assemble(kk, obuf)
                pltpu.async_copy(
                    obuf, out_hbm.at[pl.ds(base + kk * _W, _W)], sw
                )

        for obuf, sw in ((ob0, sw0), (ob1, sw1)):
            pltpu.make_async_copy(obuf, out_hbm.at[pl.ds(base, _W)], sw).wait()

    return _expand(embedding_weight, idx).reshape(b, s, d)
